# Initial kernel scaffold; baseline (speedup 1.0000x reference)
import jax
import jax.numpy as jnp
from jax.experimental import pallas as pl


def kernel(x):
    raise NotImplementedError("write your pallas kernel here")


# speedup vs baseline: 1.0810x; 1.0810x over previous
import functools
import math

import jax
import jax.numpy as jnp
from jax.experimental import pallas as pl
from jax.experimental.pallas import tpu as pltpu

R_RATIO = 0.95
MARGIN = 0.5


def _iso_body(n_blk_ref, x_blk_ref, n_all_ref, x_all_ref, z_ref):
    nb = n_blk_ref[0, 0, :]
    na = n_all_ref[0, 0, :]
    xi = x_blk_ref[0] / jnp.clip(nb[:, None], 1e-12, None)
    xa = x_all_ref[0] / jnp.clip(na[:, None], 1e-12, None)
    T = xa.shape[0]
    sim = jax.lax.dot_general(
        xi, xa, (((1,), (1,)), ((), ())),
        preferred_element_type=jnp.float32,
        precision=jax.lax.Precision.DEFAULT)
    cnt = jnp.sum(jnp.where(sim > MARGIN, 1.0, -1.0), axis=1)
    ssum = jnp.sum(sim, axis=1)
    z_ref[0, 0, :] = cnt / T + ssum / T


def _iso_pass(x, nrm, block_i=512):
    B, T, C = x.shape
    n3 = nrm.reshape(B, 1, T)
    z = pl.pallas_call(
        _iso_body,
        grid=(B, T // block_i),
        in_specs=[
            pl.BlockSpec((1, 1, block_i), lambda b, i: (b, 0, i)),
            pl.BlockSpec((1, block_i, C), lambda b, i: (b, i, 0)),
            pl.BlockSpec((1, 1, T), lambda b, i: (b, 0, 0)),
            pl.BlockSpec((1, T, C), lambda b, i: (b, 0, 0)),
        ],
        out_specs=pl.BlockSpec((1, 1, block_i), lambda b, i: (b, 0, i)),
        out_shape=jax.ShapeDtypeStruct((B, 1, T), jnp.float32),
    )(n3, x, n3, x)
    return z[:, 0, :]


def kernel(x):
    B, T, C = x.shape
    r = math.floor(T - T * R_RATIO)
    nrm = jnp.linalg.norm(x, axis=-1, keepdims=True)[..., 0]
    z = _iso_pass(x, nrm)

    iso = 1.0 - jax.nn.softmax(z, axis=-1)
    indices = jnp.argsort(iso, axis=-1)
    min_indices = indices[..., :2 * r]
    protected_idx = indices[..., 2 * r:]
    a_idx = min_indices[..., ::2]
    b_idx = min_indices[..., 1::2]

    xn = x / jnp.clip(nrm[..., None], 1e-12, None)
    xa = jnp.take_along_axis(xn, a_idx[:, :, None], axis=1)
    xb = jnp.take_along_axis(xn, b_idx[:, :, None], axis=1)
    scores = jax.lax.dot_general(
        xa, xb, (((2,), (2,)), ((0,), (0,))),
        preferred_element_type=jnp.float32,
        precision=jax.lax.Precision.DEFAULT)
    dst_idx = jnp.argmax(scores, axis=-1)
    w = iso[..., None]

    def merge_sum(t):
        prot_sorted = jnp.sort(protected_idx, axis=-1)
        protected = jnp.take_along_axis(t, prot_sorted[:, :, None], axis=1)
        src = jnp.take_along_axis(t, a_idx[:, :, None], axis=1)
        dst = jnp.take_along_axis(t, b_idx[:, :, None], axis=1)
        onehot = (dst_idx[:, :, None] == jnp.arange(r)[None, None, :]).astype(t.dtype)
        dst = dst + jnp.einsum('bik,bic->bkc', onehot, src)
        return jnp.concatenate([protected, dst], axis=1)

    xm = merge_sum(x * w)
    size = merge_sum(w)
    return xm / size, size
